# baseline (speedup 1.0000x reference)
import functools

import jax
import jax.numpy as jnp
from jax import lax
from jax.experimental import pallas as pl
from jax.experimental.pallas import tpu as pltpu
from jax.experimental.pallas import tpu_sc as plsc

NC = 2
NS = 16
CH = 128


def _round_up(a, b):
    return (a + b - 1) // b * b


def _sc_mesh():
    return plsc.VectorSubcoreMesh(core_axis_name="c", subcore_axis_name="s")


def _make_deg_kernel(NP, SCH):
    WCH = SCH // 2
    STRIPE = NP // NS

    @functools.partial(
        pl.kernel,
        mesh=_sc_mesh(),
        out_type=jax.ShapeDtypeStruct((NC, NP, 128), jnp.float32),
        scratch_types=[
            pltpu.VMEM((SCH, CH), jnp.int32),
            pltpu.VMEM((CH, 128), jnp.float32),
            pltpu.VMEM_SHARED((NP, 128), jnp.float32),
        ],
    )
    def deg_kernel(dst_hbm, ones_hbm, z128_hbm, deg_out, dst_v, ones_v, deg_sh):
        c = lax.axis_index("c")
        s = lax.axis_index("s")
        base = s * STRIPE
        pltpu.sync_copy(z128_hbm.at[pl.ds(base, STRIPE)],
                        deg_sh.at[pl.ds(base, STRIPE)])
        pltpu.sync_copy(ones_hbm, ones_v)
        pltpu.sync_copy(dst_hbm.at[s], dst_v)
        plsc.subcore_barrier()
        lo = c * WCH

        @pl.loop(0, WCH)
        def _(j):
            pltpu.sync_copy(ones_v, deg_sh.at[dst_v.at[lo + j]], add=True)

        plsc.subcore_barrier()
        pltpu.sync_copy(deg_sh.at[pl.ds(base, STRIPE)],
                        deg_out.at[c].at[pl.ds(base, STRIPE)])

    return deg_kernel


def _make_spmm_kernel(NP, SCH):
    STRIPE = NP // NS

    @functools.partial(
        pl.kernel,
        mesh=_sc_mesh(),
        out_type=jax.ShapeDtypeStruct((NC, NP, 128), jnp.float32),
        scratch_types=[
            pltpu.VMEM((16, CH), jnp.int32),
            pltpu.VMEM((16, CH), jnp.int32),
            pltpu.VMEM((CH, 128), jnp.float32),
            pltpu.VMEM((CH, 128), jnp.float32),
            pltpu.VMEM_SHARED((NP, 128), jnp.float32),
            pltpu.SemaphoreType.DMA,
            pltpu.SemaphoreType.DMA,
            pltpu.SemaphoreType.DMA,
            pltpu.SemaphoreType.DMA,
        ],
    )
    def spmm_kernel(t_hbm, src_hbm, dst_hbm, z128_hbm, agg_out,
                    src_v, dst_v, rows_a, rows_b, agg_sh, ga, gb, sa, sb):
        c = lax.axis_index("c")
        s = lax.axis_index("s")
        base = s * STRIPE
        pltpu.sync_copy(z128_hbm.at[pl.ds(base, STRIPE)],
                        agg_sh.at[pl.ds(base, STRIPE)])
        plsc.subcore_barrier()
        tc_ref = t_hbm.at[c]
        src_s = src_hbm.at[s]
        dst_s = dst_hbm.at[s]
        GC = 16

        @pl.loop(0, SCH // GC)
        def _(g):
            @pl.when(g > 0)
            def _():
                pltpu.make_async_copy(
                    rows_b, agg_sh.at[dst_v.at[GC - 1]], sb).wait()

            pltpu.sync_copy(src_s.at[pl.ds(g * GC, GC)], src_v)
            pltpu.sync_copy(dst_s.at[pl.ds(g * GC, GC)], dst_v)
            pltpu.async_copy(tc_ref.at[src_v.at[0]], rows_a, ga)

            @pl.loop(0, GC, step=2)
            def _(j):
                pltpu.make_async_copy(
                    tc_ref.at[src_v.at[j]], rows_a, ga).wait()
                pltpu.async_copy(rows_a, agg_sh.at[dst_v.at[j]], sa,
                                 add=True)

                @pl.when(j > 0)
                def _():
                    pltpu.make_async_copy(
                        rows_b, agg_sh.at[dst_v.at[j - 1]], sb).wait()

                pltpu.async_copy(tc_ref.at[src_v.at[j + 1]], rows_b, gb)
                pltpu.make_async_copy(
                    tc_ref.at[src_v.at[j + 1]], rows_b, gb).wait()
                pltpu.async_copy(rows_b, agg_sh.at[dst_v.at[j + 1]], sb,
                                 add=True)
                pltpu.make_async_copy(
                    rows_a, agg_sh.at[dst_v.at[j]], sa).wait()

                @pl.when(j + 2 < GC)
                def _():
                    pltpu.async_copy(tc_ref.at[src_v.at[j + 2]], rows_a, ga)

        pltpu.make_async_copy(
            rows_b, agg_sh.at[dst_v.at[GC - 1]], sb).wait()
        plsc.subcore_barrier()
        pltpu.sync_copy(agg_sh.at[pl.ds(base, STRIPE)],
                        agg_out.at[c].at[pl.ds(base, STRIPE)])

    return spmm_kernel


def _matmul(x_pad, w, b, NP, D, BM=512):
    def body(x_ref, w_ref, b_ref, o_ref):
        o_ref[...] = lax.dot_general(
            x_ref[...], w_ref[...], (((1,), (1,)), ((), ())),
            preferred_element_type=jnp.float32) + b_ref[...]

    return pl.pallas_call(
        body,
        grid=(NP // BM,),
        in_specs=[pl.BlockSpec((BM, D), lambda i: (i, 0)),
                  pl.BlockSpec((D, D), lambda i: (0, 0)),
                  pl.BlockSpec((1, D), lambda i: (0, 0))],
        out_specs=pl.BlockSpec((BM, D), lambda i: (i, 0)),
        out_shape=jax.ShapeDtypeStruct((NP, D), jnp.float32),
    )(x_pad, w, b.reshape(1, D))


def _prep(deg2, h0, NP, BM=512):
    def body(deg_ref, h_ref, t_ref, n2_ref, n1_ref):
        degs = deg_ref[0, :, 0:1] + deg_ref[1, :, 0:1]
        nrm = lax.rsqrt(1.0 + degs)
        h = h_ref[...]
        t_ref[0] = nrm * h[:, :128]
        t_ref[1] = nrm * h[:, 128:]
        n2_ref[...] = nrm * nrm
        n1_ref[...] = nrm

    return pl.pallas_call(
        body,
        grid=(NP // BM,),
        in_specs=[pl.BlockSpec((NC, BM, 128), lambda i: (0, i, 0)),
                  pl.BlockSpec((BM, 256), lambda i: (i, 0))],
        out_specs=[pl.BlockSpec((NC, BM, 128), lambda i: (0, i, 0)),
                   pl.BlockSpec((BM, 1), lambda i: (i, 0)),
                   pl.BlockSpec((BM, 1), lambda i: (i, 0))],
        out_shape=[jax.ShapeDtypeStruct((NC, NP, 128), jnp.float32),
                   jax.ShapeDtypeStruct((NP, 1), jnp.float32),
                   jax.ShapeDtypeStruct((NP, 1), jnp.float32)],
    )(deg2, h0)


def _ew(agg, t, n2, NP, BM=512):
    def body(agg_ref, t_ref, n2_ref, o_ref):
        n2v = n2_ref[...]
        o_ref[0] = n2v * (agg_ref[0] + t_ref[0])
        o_ref[1] = n2v * (agg_ref[1] + t_ref[1])

    return pl.pallas_call(
        body,
        grid=(NP // BM,),
        in_specs=[pl.BlockSpec((NC, BM, 128), lambda i: (0, i, 0)),
                  pl.BlockSpec((NC, BM, 128), lambda i: (0, i, 0)),
                  pl.BlockSpec((BM, 1), lambda i: (i, 0))],
        out_specs=pl.BlockSpec((NC, BM, 128), lambda i: (0, i, 0)),
        out_shape=jax.ShapeDtypeStruct((NC, NP, 128), jnp.float32),
    )(agg, t, n2)


def _fin(agg, t, n1, N, NP, BM=512):
    def body(agg_ref, t_ref, n1_ref, o_ref):
        n1v = n1_ref[...]
        o_ref[:, :128] = n1v * (agg_ref[0] + t_ref[0])
        o_ref[:, 128:] = n1v * (agg_ref[1] + t_ref[1])

    return pl.pallas_call(
        body,
        grid=(NP // BM,),
        in_specs=[pl.BlockSpec((NC, BM, 128), lambda i: (0, i, 0)),
                  pl.BlockSpec((NC, BM, 128), lambda i: (0, i, 0)),
                  pl.BlockSpec((BM, 1), lambda i: (i, 0))],
        out_specs=pl.BlockSpec((BM, 256), lambda i: (i, 0)),
        out_shape=jax.ShapeDtypeStruct((N, 256), jnp.float32),
    )(agg, t, n1)


def kernel(x, edge_index, W_xemb, b_xemb):
    N, D = x.shape
    E = edge_index.shape[1]
    NP = _round_up(N + 1, 256)
    SCH = 2 * ((E + NS * CH * 2 - 1) // (NS * CH * 2))
    EP = NS * SCH * CH
    EPAD = EP - E
    NPAD = NP - N

    src = edge_index[0]
    dst = edge_index[1]
    src_p = jnp.concatenate(
        [src, jnp.zeros((EPAD,), jnp.int32)]).reshape(NS, SCH, CH)
    pad_dst = N + (jnp.arange(EPAD, dtype=jnp.int32) % NPAD)
    dst_p = jnp.concatenate([dst, pad_dst]).reshape(NS, SCH, CH)
    x_pad = jnp.pad(x, ((0, NPAD), (0, 0)))

    z128 = jnp.zeros((NP, 128), jnp.float32)
    ones128 = jnp.ones((CH, 128), jnp.float32)

    deg_kernel = _make_deg_kernel(NP, SCH)
    spmm_kernel = _make_spmm_kernel(NP, SCH)

    deg2 = deg_kernel(dst_p, ones128, z128)
    h0 = _matmul(x_pad, W_xemb, b_xemb, NP, D)
    t, n2, n1 = _prep(deg2, h0, NP)

    agg = spmm_kernel(t, src_p, dst_p, z128)
    t = _ew(agg, t, n2, NP)
    agg = spmm_kernel(t, src_p, dst_p, z128)
    t = _ew(agg, t, n2, NP)
    agg = spmm_kernel(t, src_p, dst_p, z128)
    return _fin(agg, t, n1, N, NP)


# speedup vs baseline: 1.0812x; 1.0812x over previous
import functools

import jax
import jax.numpy as jnp
from jax import lax
from jax.experimental import pallas as pl
from jax.experimental.pallas import tpu as pltpu
from jax.experimental.pallas import tpu_sc as plsc

NC = 2
NS = 16
CH = 128


def _round_up(a, b):
    return (a + b - 1) // b * b


def _sc_mesh():
    return plsc.VectorSubcoreMesh(core_axis_name="c", subcore_axis_name="s")


def _make_deg_kernel(NP, SCH):
    WCH = SCH // 2
    STRIPE = NP // NS

    @functools.partial(
        pl.kernel,
        mesh=_sc_mesh(),
        out_type=jax.ShapeDtypeStruct((NC, NP, 128), jnp.float32),
        scratch_types=[
            pltpu.VMEM((SCH, CH), jnp.int32),
            pltpu.VMEM((CH, 128), jnp.float32),
            pltpu.VMEM_SHARED((NP, 128), jnp.float32),
        ],
    )
    def deg_kernel(dst_hbm, ones_hbm, z128_hbm, deg_out, dst_v, ones_v, deg_sh):
        c = lax.axis_index("c")
        s = lax.axis_index("s")
        base = s * STRIPE
        pltpu.sync_copy(z128_hbm.at[pl.ds(base, STRIPE)],
                        deg_sh.at[pl.ds(base, STRIPE)])
        pltpu.sync_copy(ones_hbm, ones_v)
        pltpu.sync_copy(dst_hbm.at[s], dst_v)
        plsc.subcore_barrier()
        lo = c * WCH

        @pl.loop(0, WCH)
        def _(j):
            pltpu.sync_copy(ones_v, deg_sh.at[dst_v.at[lo + j]], add=True)

        plsc.subcore_barrier()
        pltpu.sync_copy(deg_sh.at[pl.ds(base, STRIPE)],
                        deg_out.at[c].at[pl.ds(base, STRIPE)])

    return deg_kernel


def _make_spmm_kernel(NP, SCH):
    STRIPE = NP // NS

    @functools.partial(
        pl.kernel,
        mesh=_sc_mesh(),
        out_type=jax.ShapeDtypeStruct((NC, NP, 128), jnp.float32),
        scratch_types=[
            pltpu.VMEM((40, CH), jnp.int32),
            pltpu.VMEM((40, CH), jnp.int32),
            pltpu.VMEM((CH, 128), jnp.float32),
            pltpu.VMEM((CH, 128), jnp.float32),
            pltpu.VMEM_SHARED((NP, 128), jnp.float32),
            pltpu.SemaphoreType.DMA,
            pltpu.SemaphoreType.DMA,
            pltpu.SemaphoreType.DMA,
            pltpu.SemaphoreType.DMA,
        ],
    )
    def spmm_kernel(t_hbm, src_hbm, dst_hbm, z128_hbm, agg_out,
                    src_v, dst_v, rows_a, rows_b, agg_sh, ga, gb, sa, sb):
        c = lax.axis_index("c")
        s = lax.axis_index("s")
        base = s * STRIPE
        pltpu.sync_copy(z128_hbm.at[pl.ds(base, STRIPE)],
                        agg_sh.at[pl.ds(base, STRIPE)])
        plsc.subcore_barrier()
        tc_ref = t_hbm.at[c]
        src_s = src_hbm.at[s]
        dst_s = dst_hbm.at[s]
        GC = 40

        @pl.loop(0, SCH // GC)
        def _(g):
            pltpu.sync_copy(src_s.at[pl.ds(g * GC, GC)], src_v)
            pltpu.sync_copy(dst_s.at[pl.ds(g * GC, GC)], dst_v)
            pltpu.async_copy(tc_ref.at[src_v.at[0]], rows_a, ga)

            @pl.loop(0, GC, step=2)
            def _(j):
                pltpu.async_copy(tc_ref.at[src_v.at[j + 1]], rows_b, gb)
                pltpu.make_async_copy(
                    tc_ref.at[src_v.at[j]], rows_a, ga).wait()
                pltpu.sync_copy(rows_a, agg_sh.at[dst_v.at[j]], add=True)

                @pl.when(j + 2 < GC)
                def _():
                    pltpu.async_copy(tc_ref.at[src_v.at[j + 2]], rows_a, ga)

                pltpu.make_async_copy(
                    tc_ref.at[src_v.at[j + 1]], rows_b, gb).wait()
                pltpu.sync_copy(rows_b, agg_sh.at[dst_v.at[j + 1]], add=True)

        plsc.subcore_barrier()
        pltpu.sync_copy(agg_sh.at[pl.ds(base, STRIPE)],
                        agg_out.at[c].at[pl.ds(base, STRIPE)])

    return spmm_kernel


def _matmul(x_pad, w, b, NP, D, BM=512):
    def body(x_ref, w_ref, b_ref, o_ref):
        o_ref[...] = lax.dot_general(
            x_ref[...], w_ref[...], (((1,), (1,)), ((), ())),
            preferred_element_type=jnp.float32) + b_ref[...]

    return pl.pallas_call(
        body,
        grid=(NP // BM,),
        in_specs=[pl.BlockSpec((BM, D), lambda i: (i, 0)),
                  pl.BlockSpec((D, D), lambda i: (0, 0)),
                  pl.BlockSpec((1, D), lambda i: (0, 0))],
        out_specs=pl.BlockSpec((BM, D), lambda i: (i, 0)),
        out_shape=jax.ShapeDtypeStruct((NP, D), jnp.float32),
    )(x_pad, w, b.reshape(1, D))


def _prep(deg2, h0, NP, BM=512):
    def body(deg_ref, h_ref, t_ref, n2_ref, n1_ref):
        degs = deg_ref[0, :, 0:1] + deg_ref[1, :, 0:1]
        nrm = lax.rsqrt(1.0 + degs)
        h = h_ref[...]
        t_ref[0] = nrm * h[:, :128]
        t_ref[1] = nrm * h[:, 128:]
        n2_ref[...] = nrm * nrm
        n1_ref[...] = nrm

    return pl.pallas_call(
        body,
        grid=(NP // BM,),
        in_specs=[pl.BlockSpec((NC, BM, 128), lambda i: (0, i, 0)),
                  pl.BlockSpec((BM, 256), lambda i: (i, 0))],
        out_specs=[pl.BlockSpec((NC, BM, 128), lambda i: (0, i, 0)),
                   pl.BlockSpec((BM, 1), lambda i: (i, 0)),
                   pl.BlockSpec((BM, 1), lambda i: (i, 0))],
        out_shape=[jax.ShapeDtypeStruct((NC, NP, 128), jnp.float32),
                   jax.ShapeDtypeStruct((NP, 1), jnp.float32),
                   jax.ShapeDtypeStruct((NP, 1), jnp.float32)],
    )(deg2, h0)


def _ew(agg, t, n2, NP, BM=512):
    def body(agg_ref, t_ref, n2_ref, o_ref):
        n2v = n2_ref[...]
        o_ref[0] = n2v * (agg_ref[0] + t_ref[0])
        o_ref[1] = n2v * (agg_ref[1] + t_ref[1])

    return pl.pallas_call(
        body,
        grid=(NP // BM,),
        in_specs=[pl.BlockSpec((NC, BM, 128), lambda i: (0, i, 0)),
                  pl.BlockSpec((NC, BM, 128), lambda i: (0, i, 0)),
                  pl.BlockSpec((BM, 1), lambda i: (i, 0))],
        out_specs=pl.BlockSpec((NC, BM, 128), lambda i: (0, i, 0)),
        out_shape=jax.ShapeDtypeStruct((NC, NP, 128), jnp.float32),
    )(agg, t, n2)


def _fin(agg, t, n1, N, NP, BM=512):
    def body(agg_ref, t_ref, n1_ref, o_ref):
        n1v = n1_ref[...]
        o_ref[:, :128] = n1v * (agg_ref[0] + t_ref[0])
        o_ref[:, 128:] = n1v * (agg_ref[1] + t_ref[1])

    return pl.pallas_call(
        body,
        grid=(NP // BM,),
        in_specs=[pl.BlockSpec((NC, BM, 128), lambda i: (0, i, 0)),
                  pl.BlockSpec((NC, BM, 128), lambda i: (0, i, 0)),
                  pl.BlockSpec((BM, 1), lambda i: (i, 0))],
        out_specs=pl.BlockSpec((BM, 256), lambda i: (i, 0)),
        out_shape=jax.ShapeDtypeStruct((N, 256), jnp.float32),
    )(agg, t, n1)


def kernel(x, edge_index, W_xemb, b_xemb):
    N, D = x.shape
    E = edge_index.shape[1]
    NP = _round_up(N + 1, 256)
    SCH = 2 * ((E + NS * CH * 2 - 1) // (NS * CH * 2))
    EP = NS * SCH * CH
    EPAD = EP - E
    NPAD = NP - N

    src = edge_index[0]
    dst = edge_index[1]
    src_p = jnp.concatenate(
        [src, jnp.zeros((EPAD,), jnp.int32)]).reshape(NS, SCH, CH)
    pad_dst = N + (jnp.arange(EPAD, dtype=jnp.int32) % NPAD)
    dst_p = jnp.concatenate([dst, pad_dst]).reshape(NS, SCH, CH)
    x_pad = jnp.pad(x, ((0, NPAD), (0, 0)))

    z128 = jnp.zeros((NP, 128), jnp.float32)
    ones128 = jnp.ones((CH, 128), jnp.float32)

    deg_kernel = _make_deg_kernel(NP, SCH)
    spmm_kernel = _make_spmm_kernel(NP, SCH)

    deg2 = deg_kernel(dst_p, ones128, z128)
    h0 = _matmul(x_pad, W_xemb, b_xemb, NP, D)
    t, n2, n1 = _prep(deg2, h0, NP)

    agg = spmm_kernel(t, src_p, dst_p, z128)
    t = _ew(agg, t, n2, NP)
    agg = spmm_kernel(t, src_p, dst_p, z128)
    t = _ew(agg, t, n2, NP)
    agg = spmm_kernel(t, src_p, dst_p, z128)
    return _fin(agg, t, n1, N, NP)
